# baseline (speedup 1.0000x reference)
import functools

import jax
import jax.numpy as jnp
from jax import lax
from jax.experimental import pallas as pl
from jax.experimental.pallas import tpu as pltpu
from jax.experimental.pallas import tpu_sc as plsc

_K = 8
_NEG = -3.0e38


def _logits_body(x_ref, w_ref, logits_ref):
    logits_ref[...] = lax.dot_general(
        x_ref[...], w_ref[...],
        dimension_numbers=(((1,), (1,)), ((), ())),
        preferred_element_type=jnp.float32)


def _logits_body3d(x_ref, w_ref, logits_ref):
    logits_ref[...] = lax.dot_general(
        x_ref[0], w_ref[...],
        dimension_numbers=(((1,), (1,)), ((), ())),
        preferred_element_type=jnp.float32)[None]


def _logits_tc(x3d, centroids):
    b, s, h = x3d.shape
    e_dim = centroids.shape[0]
    bm = 1024
    nbs = s // bm
    return pl.pallas_call(
        _logits_body3d,
        grid=(b * nbs,),
        in_specs=[
            pl.BlockSpec((1, bm, h), lambda i: (i // nbs, i % nbs, 0)),
            pl.BlockSpec((e_dim, h), lambda i: (0, 0)),
        ],
        out_specs=pl.BlockSpec((1, bm, e_dim), lambda i: (i // nbs, i % nbs, 0)),
        out_shape=jax.ShapeDtypeStruct((b, s, e_dim), jnp.float32),
        compiler_params=pltpu.CompilerParams(
            dimension_semantics=("parallel",)),
    )(x3d, centroids)


def _make_routing_sc(b, s, e_dim):
    n = b * s
    info = plsc.get_sparse_core_info()
    nc, ns, lanes = info.num_cores, info.num_subcores, info.num_lanes
    nw = nc * ns
    assert n % (nw * lanes) == 0 and e_dim % lanes == 0
    c_per = n // nw
    assert s % c_per == 0
    ngroups = c_per // lanes
    mesh = plsc.VectorSubcoreMesh(core_axis_name="c", subcore_axis_name="s")

    @functools.partial(
        pl.kernel,
        mesh=mesh,
        out_type=[
            jax.ShapeDtypeStruct((b, s, e_dim), jnp.float32),
            jax.ShapeDtypeStruct((b, s, _K), jnp.int32),
        ],
        scratch_types=[
            pltpu.VMEM((c_per, e_dim), jnp.float32),
            pltpu.VMEM((c_per, e_dim), jnp.float32),
            pltpu.VMEM((c_per, _K), jnp.int32),
            pltpu.SemaphoreType.DMA,
        ],
        compiler_params=pltpu.CompilerParams(needs_layout_passes=False),
    )
    def routing(logits_hbm, gate_hbm, idx_hbm, lg_v, gate_v, idx_v, sem):
        wid = lax.axis_index("s") * nc + lax.axis_index("c")
        base = wid * c_per
        bi = base // s
        so = base % s
        copy = pltpu.async_copy(
            logits_hbm.at[bi, pl.ds(so, c_per)], lg_v, sem)

        nchunk = e_dim // lanes

        def zero_body(r, carry):
            for cch in range(nchunk):
                gate_v[r, pl.ds(cch * lanes, lanes)] = (
                    jnp.zeros((lanes,), jnp.float32))
            return carry
        lax.fori_loop(0, c_per, zero_body, 0, unroll=2)

        copy.wait()

        lane_iota = jnp.arange(lanes, dtype=jnp.int32)
        ninter = 2

        def group_body(g, carry):
            rows = [(ninter * g + t) * lanes + lane_iota for t in range(ninter)]

            def ins_body(e, st):
                out = []
                for t in range(ninter):
                    v = list(st[2 * _K * t:2 * _K * t + _K])
                    ix = list(st[2 * _K * t + _K:2 * _K * (t + 1)])
                    col = jnp.full((lanes,), e, jnp.int32)
                    val = plsc.load_gather(lg_v, [rows[t], col])
                    c = [val > v[j] for j in range(_K)]
                    nv = [jnp.where(c[0], val, v[0])]
                    ni = [jnp.where(c[0], col, ix[0])]
                    for j in range(1, _K):
                        nv.append(jnp.where(c[j - 1], v[j - 1],
                                            jnp.where(c[j], val, v[j])))
                        ni.append(jnp.where(c[j - 1], ix[j - 1],
                                            jnp.where(c[j], col, ix[j])))
                    out.extend(nv + ni)
                return tuple(out)

            neg = jnp.full((lanes,), _NEG, jnp.float32)
            zero_i = jnp.zeros((lanes,), jnp.int32)
            st = lax.fori_loop(
                0, e_dim, ins_body,
                (*([neg] * _K), *([zero_i] * _K)) * ninter,
                unroll=2)

            for t in range(ninter):
                v = st[2 * _K * t:2 * _K * t + _K]
                ix = st[2 * _K * t + _K:2 * _K * (t + 1)]
                m = v[0]

                exps = [jnp.exp(v[j] - m) for j in range(_K)]
                t_sum = exps[0]
                for j in range(1, _K):
                    t_sum = t_sum + exps[j]
                denom = t_sum

                for j in range(_K):
                    plsc.store_scatter(gate_v, [rows[t], ix[j]],
                                       exps[j] / denom)
                    plsc.store_scatter(
                        idx_v, [rows[t], jnp.full((lanes,), j, jnp.int32)],
                        ix[j])
            return carry

        lax.fori_loop(0, ngroups // ninter, group_body, 0)

        pltpu.sync_copy(gate_v, gate_hbm.at[bi, pl.ds(so, c_per)])
        pltpu.sync_copy(idx_v, idx_hbm.at[bi, pl.ds(so, c_per)])

    return routing


def kernel(hidden_states, expert_centroids):
    b, s, h = hidden_states.shape
    e_dim = expert_centroids.shape[0]
    logits = _logits_tc(hidden_states, expert_centroids)
    gate, idx = _make_routing_sc(b, s, e_dim)(logits)
    return (gate, idx, logits)


# speedup vs baseline: 1.0199x; 1.0199x over previous
import functools

import jax
import jax.numpy as jnp
from jax import lax
from jax.experimental import pallas as pl
from jax.experimental.pallas import tpu as pltpu
from jax.experimental.pallas import tpu_sc as plsc

_K = 8
_NEG = -3.0e38


def _logits_body(x_ref, w_ref, logits_ref):
    logits_ref[...] = lax.dot_general(
        x_ref[...], w_ref[...],
        dimension_numbers=(((1,), (1,)), ((), ())),
        preferred_element_type=jnp.float32)


def _logits_body3d(x_ref, w_ref, logits_ref):
    logits_ref[...] = lax.dot_general(
        x_ref[0], w_ref[...],
        dimension_numbers=(((1,), (1,)), ((), ())),
        preferred_element_type=jnp.float32)[None]


def _logits_tc(x3d, centroids):
    b, s, h = x3d.shape
    e_dim = centroids.shape[0]
    bm = 1024
    nbs = s // bm
    return pl.pallas_call(
        _logits_body3d,
        grid=(b * nbs,),
        in_specs=[
            pl.BlockSpec((1, bm, h), lambda i: (i // nbs, i % nbs, 0)),
            pl.BlockSpec((e_dim, h), lambda i: (0, 0)),
        ],
        out_specs=pl.BlockSpec((1, bm, e_dim), lambda i: (i // nbs, i % nbs, 0)),
        out_shape=jax.ShapeDtypeStruct((b, s, e_dim), jnp.float32),
        compiler_params=pltpu.CompilerParams(
            dimension_semantics=("parallel",)),
    )(x3d, centroids)


def _make_routing_sc(b, s, e_dim):
    n = b * s
    info = plsc.get_sparse_core_info()
    nc, ns, lanes = info.num_cores, info.num_subcores, info.num_lanes
    nw = nc * ns
    assert n % (nw * lanes) == 0 and e_dim % lanes == 0
    c_per = n // nw
    assert s % c_per == 0
    ngroups = c_per // lanes
    mesh = plsc.VectorSubcoreMesh(core_axis_name="c", subcore_axis_name="s")

    @functools.partial(
        pl.kernel,
        mesh=mesh,
        out_type=[
            jax.ShapeDtypeStruct((b, s, e_dim), jnp.float32),
            jax.ShapeDtypeStruct((b, s, _K), jnp.int32),
        ],
        scratch_types=[
            pltpu.VMEM((c_per, e_dim), jnp.float32),
            pltpu.VMEM((c_per, e_dim), jnp.float32),
            pltpu.VMEM((c_per, _K), jnp.int32),
            pltpu.SemaphoreType.DMA,
        ],
        compiler_params=pltpu.CompilerParams(needs_layout_passes=False),
    )
    def routing(logits_hbm, gate_hbm, idx_hbm, lg_v, gate_v, idx_v, sem):
        wid = lax.axis_index("s") * nc + lax.axis_index("c")
        base = wid * c_per
        bi = base // s
        so = base % s
        copy = pltpu.async_copy(
            logits_hbm.at[bi, pl.ds(so, c_per)], lg_v, sem)

        nchunk = e_dim // lanes

        def zero_body(r, carry):
            for cch in range(nchunk):
                gate_v[r, pl.ds(cch * lanes, lanes)] = (
                    jnp.zeros((lanes,), jnp.float32))
            return carry
        lax.fori_loop(0, c_per, zero_body, 0, unroll=4)

        copy.wait()

        lane_iota = jnp.arange(lanes, dtype=jnp.int32)
        ninter = 2

        def group_body(g, carry):
            rows = [(ninter * g + t) * lanes + lane_iota for t in range(ninter)]

            def ins_body(e, st):
                out = []
                for t in range(ninter):
                    v = list(st[2 * _K * t:2 * _K * t + _K])
                    ix = list(st[2 * _K * t + _K:2 * _K * (t + 1)])
                    col = jnp.full((lanes,), e, jnp.int32)
                    val = plsc.load_gather(lg_v, [rows[t], col])
                    c = [val > v[j] for j in range(_K)]
                    nv = [jnp.where(c[0], val, v[0])]
                    ni = [jnp.where(c[0], col, ix[0])]
                    for j in range(1, _K):
                        nv.append(jnp.where(c[j - 1], v[j - 1],
                                            jnp.where(c[j], val, v[j])))
                        ni.append(jnp.where(c[j - 1], ix[j - 1],
                                            jnp.where(c[j], col, ix[j])))
                    out.extend(nv + ni)
                return tuple(out)

            neg = jnp.full((lanes,), _NEG, jnp.float32)
            zero_i = jnp.zeros((lanes,), jnp.int32)
            st = lax.fori_loop(
                0, e_dim, ins_body,
                (*([neg] * _K), *([zero_i] * _K)) * ninter,
                unroll=8)

            for t in range(ninter):
                v = st[2 * _K * t:2 * _K * t + _K]
                ix = st[2 * _K * t + _K:2 * _K * (t + 1)]
                m = v[0]

                exps = [jnp.exp(v[j] - m) for j in range(_K)]
                t_sum = exps[0]
                for j in range(1, _K):
                    t_sum = t_sum + exps[j]
                denom = t_sum

                for j in range(_K):
                    plsc.store_scatter(gate_v, [rows[t], ix[j]],
                                       exps[j] / denom)
                    plsc.store_scatter(
                        idx_v, [rows[t], jnp.full((lanes,), j, jnp.int32)],
                        ix[j])
            return carry

        lax.fori_loop(0, ngroups // ninter, group_body, 0)

        pltpu.sync_copy(gate_v, gate_hbm.at[bi, pl.ds(so, c_per)])
        pltpu.sync_copy(idx_v, idx_hbm.at[bi, pl.ds(so, c_per)])

    return routing


def kernel(hidden_states, expert_centroids):
    b, s, h = hidden_states.shape
    e_dim = expert_centroids.shape[0]
    logits = _logits_tc(hidden_states, expert_centroids)
    gate, idx = _make_routing_sc(b, s, e_dim)(logits)
    return (gate, idx, logits)


# speedup vs baseline: 1.0210x; 1.0010x over previous
import functools

import jax
import jax.numpy as jnp
from jax import lax
from jax.experimental import pallas as pl
from jax.experimental.pallas import tpu as pltpu
from jax.experimental.pallas import tpu_sc as plsc

_K = 8
_NEG = -3.0e38


def _logits_body3d(x_ref, w_ref, logits_ref):
    logits_ref[...] = lax.dot_general(
        x_ref[0], w_ref[...],
        dimension_numbers=(((1,), (1,)), ((), ())),
        preferred_element_type=jnp.float32)[None]


def _logits_tc(x3d, centroids):
    b, s, h = x3d.shape
    e_dim = centroids.shape[0]
    bm = 1024
    nbs = s // bm
    return pl.pallas_call(
        _logits_body3d,
        grid=(b * nbs,),
        in_specs=[
            pl.BlockSpec((1, bm, h), lambda i: (i // nbs, i % nbs, 0)),
            pl.BlockSpec((e_dim, h), lambda i: (0, 0)),
        ],
        out_specs=pl.BlockSpec((1, bm, e_dim), lambda i: (i // nbs, i % nbs, 0)),
        out_shape=jax.ShapeDtypeStruct((b, s, e_dim), jnp.float32),
        compiler_params=pltpu.CompilerParams(
            dimension_semantics=("parallel",)),
    )(x3d, centroids)


def _make_routing_sc(b, s, e_dim):
    n = b * s
    info = plsc.get_sparse_core_info()
    nc, ns, lanes = info.num_cores, info.num_subcores, info.num_lanes
    nw = nc * ns
    assert n % (nw * lanes) == 0 and e_dim % lanes == 0
    c_per = n // nw
    assert s % c_per == 0
    ngroups = c_per // lanes
    mesh = plsc.VectorSubcoreMesh(core_axis_name="c", subcore_axis_name="s")

    @functools.partial(
        pl.kernel,
        mesh=mesh,
        out_type=[
            jax.ShapeDtypeStruct((b, s, e_dim), jnp.float32),
            jax.ShapeDtypeStruct((b, s, _K), jnp.int32),
        ],
        scratch_types=[
            pltpu.VMEM((c_per, e_dim), jnp.float32),
            pltpu.VMEM((c_per, e_dim), jnp.float32),
            pltpu.VMEM((c_per, _K), jnp.int32),
            pltpu.SemaphoreType.DMA,
        ],
        compiler_params=pltpu.CompilerParams(needs_layout_passes=False),
    )
    def routing(logits_hbm, gate_hbm, idx_hbm, lg_v, gate_v, idx_v, sem):
        wid = lax.axis_index("s") * nc + lax.axis_index("c")
        base = wid * c_per
        bi = base // s
        so = base % s
        copy = pltpu.async_copy(
            logits_hbm.at[bi, pl.ds(so, c_per)], lg_v, sem)

        nchunk = e_dim // lanes

        def zero_body(r, carry):
            for cch in range(nchunk):
                gate_v[r, pl.ds(cch * lanes, lanes)] = (
                    jnp.zeros((lanes,), jnp.float32))
            return carry
        lax.fori_loop(0, c_per, zero_body, 0, unroll=4)

        copy.wait()

        lane_iota = jnp.arange(lanes, dtype=jnp.int32)
        ninter = 2

        def group_body(g, carry):
            rows = [(ninter * g + t) * lanes + lane_iota for t in range(ninter)]

            def ins_body(e, st):
                out = []
                for t in range(ninter):
                    v = list(st[2 * _K * t:2 * _K * t + _K])
                    ix = list(st[2 * _K * t + _K:2 * _K * (t + 1)])
                    col = jnp.full((lanes,), e, jnp.int32)
                    val = plsc.load_gather(lg_v, [rows[t], col])
                    c = [val > v[j] for j in range(_K)]
                    nv = [jnp.where(c[0], val, v[0])]
                    ni = [jnp.where(c[0], col, ix[0])]
                    for j in range(1, _K):
                        nv.append(jnp.where(c[j - 1], v[j - 1],
                                            jnp.where(c[j], val, v[j])))
                        ni.append(jnp.where(c[j - 1], ix[j - 1],
                                            jnp.where(c[j], col, ix[j])))
                    out.extend(nv + ni)
                return tuple(out)

            neg = jnp.full((lanes,), _NEG, jnp.float32)
            zero_i = jnp.zeros((lanes,), jnp.int32)
            st = lax.fori_loop(
                0, e_dim, ins_body,
                (*([neg] * _K), *([zero_i] * _K)) * ninter,
                unroll=8)

            for t in range(ninter):
                v = st[2 * _K * t:2 * _K * t + _K]
                ix = st[2 * _K * t + _K:2 * _K * (t + 1)]
                m = v[0]

                exps = [jnp.exp(v[j] - m) for j in range(_K)]
                t_sum = exps[0]
                for j in range(1, _K):
                    t_sum = t_sum + exps[j]
                denom = t_sum

                for j in range(_K):
                    plsc.store_scatter(gate_v, [rows[t], ix[j]],
                                       exps[j] / denom)
                    plsc.store_scatter(
                        idx_v, [rows[t], jnp.full((lanes,), j, jnp.int32)],
                        ix[j])
            return carry

        lax.fori_loop(0, ngroups // ninter, group_body, 0)

        pltpu.sync_copy(gate_v, gate_hbm.at[bi, pl.ds(so, c_per)])
        pltpu.sync_copy(idx_v, idx_hbm.at[bi, pl.ds(so, c_per)])

    return routing


def kernel(hidden_states, expert_centroids):
    b, s, h = hidden_states.shape
    e_dim = expert_centroids.shape[0]
    logits = _logits_tc(hidden_states, expert_centroids)
    gate, idx = _make_routing_sc(b, s, e_dim)(logits)
    return (gate, idx, logits)
